# baseline (speedup 1.0000x reference)
import functools

import jax
import jax.numpy as jnp
from jax import lax
from jax.experimental import pallas as pl
from jax.experimental.pallas import tpu as pltpu
from jax.experimental.pallas import tpu_sc as plsc

N = 320000
M = 10000
D = 128
MP = 10240

NC, NS = 2, 16
PER_TILE = N // (NC * NS)
CHUNK = 64
NBUF = 4
NFULL = PER_TILE // CHUNK
TAIL = PER_TILE - NFULL * CHUNK
OUT_ROWS = MP // NS
CROWS = 8


def _mm_relu_body(x_ref, w_ref, b_ref, o_ref):
    x = x_ref[...].astype(jnp.bfloat16)
    w = w_ref[...].astype(jnp.bfloat16)
    o_ref[...] = jnp.maximum(
        jnp.dot(x, w, preferred_element_type=jnp.float32) + b_ref[...], 0.0)


def _node_transform(hyper_node, W_v2e, b_v2e):
    BN = 6400
    grid = (N // BN,)
    return pl.pallas_call(
        _mm_relu_body,
        grid=grid,
        in_specs=[
            pl.BlockSpec((BN, D), lambda i: (i, 0)),
            pl.BlockSpec((D, D), lambda i: (0, 0)),
            pl.BlockSpec((1, D), lambda i: (0, 0)),
        ],
        out_specs=pl.BlockSpec((BN, D), lambda i: (i, 0)),
        out_shape=jax.ShapeDtypeStruct((N, D), jnp.float32),
    )(hyper_node, W_v2e, b_v2e.reshape(1, D))


def _scatter_body(ni_hbm, idx_hbm, zsum_hbm,
                  sum_out, cnt_out,
                  rows_v0, rows_v1, rows_v2, rows_v3,
                  idx_v0, idx_v1, idx_v2, idx_v3, idxt_v,
                  ones_v, zc_v, sum_acc, cnt_acc,
                  rsem0, rsem1, rsem2, rsem3,
                  isem0, isem1, isem2, isem3,
                  ssem0, ssem1, ssem2, ssem3,
                  csem0, csem1, csem2, csem3):
    c = lax.axis_index("c")
    s = lax.axis_index("s")
    rows = (rows_v0, rows_v1, rows_v2, rows_v3)
    idxs = (idx_v0, idx_v1, idx_v2, idx_v3)
    rsem = (rsem0, rsem1, rsem2, rsem3)
    isem = (isem0, isem1, isem2, isem3)
    ssem = (ssem0, ssem1, ssem2, ssem3)
    csem = (csem0, csem1, csem2, csem3)

    one16 = jnp.ones((16,), jnp.float32)
    zero16 = jnp.zeros((16,), jnp.float32)
    for k in range(CHUNK // 16):
        ones_v[pl.ds(16 * k, 16)] = one16
    for k in range(OUT_ROWS // 16):
        zc_v[pl.ds(16 * k, 16)] = zero16

    r0 = pl.multiple_of(s * OUT_ROWS, 8)
    pltpu.sync_copy(zsum_hbm.at[pl.ds(r0, OUT_ROWS)],
                    sum_acc.at[pl.ds(r0, OUT_ROWS)])
    pltpu.sync_copy(zc_v, cnt_acc.at[pl.ds(r0, OUT_ROWS)])
    plsc.subcore_barrier()

    node_base = c * (N // NC) + s * PER_TILE

    def start_fetch(k, b):
        kk = jnp.where(k < NFULL, k, 0)
        base = pl.multiple_of(node_base + kk * CHUNK, 8)
        pltpu.make_async_copy(ni_hbm.at[pl.ds(base, CHUNK)],
                              rows[b], rsem[b]).start()
        pltpu.make_async_copy(idx_hbm.at[pl.ds(base, CHUNK)],
                              idxs[b], isem[b]).start()

    def wait_fetch(b):
        pltpu.make_async_copy(ni_hbm.at[pl.ds(0, CHUNK)],
                              rows[b], rsem[b]).wait()
        pltpu.make_async_copy(idx_hbm.at[pl.ds(0, CHUNK)],
                              idxs[b], isem[b]).wait()

    def start_scat(b):
        pltpu.make_async_copy(rows[b], sum_acc.at[idxs[b]],
                              ssem[b]).start(add=True)
        pltpu.make_async_copy(ones_v, cnt_acc.at[idxs[b]],
                              csem[b]).start(add=True)

    def wait_scat(b):
        pltpu.make_async_copy(rows[b], sum_acc.at[idxs[b]], ssem[b]).wait()
        pltpu.make_async_copy(ones_v, cnt_acc.at[idxs[b]], csem[b]).wait()

    def step(k, b, first):
        wait_fetch(b)
        start_scat(b)
        if not first:
            wait_scat((b - 2) % NBUF)
        start_fetch(k + 2, (b + 2) % NBUF)

    start_fetch(0, 0)
    start_fetch(1, 1)
    step(0, 0, True)
    step(1, 1, True)

    @pl.loop(0, (NFULL - 2) // NBUF)
    def body(g):
        k = NBUF * g + 2
        step(k, 2, False)
        step(k + 1, 3, False)
        step(k + 2, 0, False)
        step(k + 3, 1, False)

    rem = (NFULL - 2) % NBUF
    for i in range(rem):
        step(NFULL - rem + i, (NFULL - rem + i) % NBUF, False)

    wait_scat((NFULL - 2) % NBUF)
    wait_scat((NFULL - 1) % NBUF)
    wait_fetch(NFULL % NBUF)
    wait_fetch((NFULL + 1) % NBUF)

    tbase = pl.multiple_of(node_base + NFULL * CHUNK, 8)
    pltpu.sync_copy(ni_hbm.at[pl.ds(tbase, TAIL)], rows_v0.at[pl.ds(0, TAIL)])
    pltpu.sync_copy(idx_hbm.at[pl.ds(tbase, TAIL)], idxt_v)
    pltpu.sync_copy(rows_v0.at[pl.ds(0, TAIL)], sum_acc.at[idxt_v], add=True)
    pltpu.sync_copy(ones_v.at[pl.ds(0, TAIL)], cnt_acc.at[idxt_v], add=True)

    plsc.subcore_barrier()

    pltpu.sync_copy(sum_acc.at[pl.ds(r0, OUT_ROWS)],
                    sum_out.at[c, pl.ds(r0, OUT_ROWS)])
    pltpu.sync_copy(cnt_acc.at[pl.ds(r0, OUT_ROWS)], zc_v)
    for k in range(OUT_ROWS // 16):
        rows_v0[k // 8, pl.ds((k % 8) * 16, 16)] = zc_v[pl.ds(16 * k, 16)]
    cr0 = pl.multiple_of(s * CROWS, 8)
    pltpu.sync_copy(rows_v0.at[pl.ds(0, CROWS)],
                    cnt_out.at[c, pl.ds(cr0, CROWS)])


def _segment_sums(node_info, idx, zsum):
    mesh = plsc.VectorSubcoreMesh(core_axis_name="c", subcore_axis_name="s")
    f = functools.partial(
        pl.kernel,
        out_type=(jax.ShapeDtypeStruct((NC, MP, D), jnp.float32),
                  jax.ShapeDtypeStruct((NC, NS * CROWS, D), jnp.float32)),
        mesh=mesh,
        scratch_types=[
            pltpu.VMEM((CHUNK, D), jnp.float32),
            pltpu.VMEM((CHUNK, D), jnp.float32),
            pltpu.VMEM((CHUNK, D), jnp.float32),
            pltpu.VMEM((CHUNK, D), jnp.float32),
            pltpu.VMEM((CHUNK,), jnp.int32),
            pltpu.VMEM((CHUNK,), jnp.int32),
            pltpu.VMEM((CHUNK,), jnp.int32),
            pltpu.VMEM((CHUNK,), jnp.int32),
            pltpu.VMEM((TAIL,), jnp.int32),
            pltpu.VMEM((CHUNK,), jnp.float32),
            pltpu.VMEM((OUT_ROWS,), jnp.float32),
            pltpu.VMEM_SHARED((MP, D), jnp.float32),
            pltpu.VMEM_SHARED((MP,), jnp.float32),
        ] + [pltpu.SemaphoreType.DMA for _ in range(16)],
    )(_scatter_body)
    return f(node_info, idx, zsum)


def _update_body(sums_ref, cnt_ref, he_ref, wa_ref, wb_ref, b_ref, o_ref):
    ssum = sums_ref[0] + sums_ref[1]
    cnt = jnp.sum(cnt_ref[...], axis=1, keepdims=True)
    mean = ssum / jnp.maximum(cnt, 1.0)
    h = jnp.dot(mean, wa_ref[...], preferred_element_type=jnp.float32)
    h = h + jnp.dot(he_ref[...], wb_ref[...], preferred_element_type=jnp.float32)
    h = jnp.maximum(h + b_ref[...], 0.0)
    nrm = jnp.sqrt(jnp.sum(h * h, axis=-1, keepdims=True))
    o_ref[...] = h / jnp.maximum(nrm, 1e-12)


def _update(sums, cntsT, hyperedge, W_upd, b_upd):
    BR = 2000
    grid = (M // BR,)
    return pl.pallas_call(
        _update_body,
        grid=grid,
        in_specs=[
            pl.BlockSpec((NC, BR, D), lambda i: (0, i, 0)),
            pl.BlockSpec((BR, NC), lambda i: (i, 0)),
            pl.BlockSpec((BR, D), lambda i: (i, 0)),
            pl.BlockSpec((D, D), lambda i: (0, 0)),
            pl.BlockSpec((D, D), lambda i: (0, 0)),
            pl.BlockSpec((1, D), lambda i: (0, 0)),
        ],
        out_specs=pl.BlockSpec((BR, D), lambda i: (i, 0)),
        out_shape=jax.ShapeDtypeStruct((M, D), jnp.float32),
    )(sums, cntsT, hyperedge, W_upd[:D], W_upd[D:], b_upd.reshape(1, D))


def kernel(hyperedge, hyper_node, ve_affiliation, W_v2e, b_v2e, W_upd, b_upd):
    assert hyper_node.shape == (N, D) and hyperedge.shape == (M, D)
    node_info = _node_transform(hyper_node, W_v2e, b_v2e)
    idx = ve_affiliation[0]
    zsum = jnp.zeros((MP, D), jnp.float32)
    sums, cnts_wide = _segment_sums(node_info, idx, zsum)
    cntsT = (cnts_wide.reshape(NC, NS, CROWS * D)[:, :, :OUT_ROWS]
             .reshape(NC, MP).T)
    return _update(sums, cntsT, hyperedge, W_upd, b_upd)


# speedup vs baseline: 1.0683x; 1.0683x over previous
import functools

import jax
import jax.numpy as jnp
from jax import lax
from jax.experimental import pallas as pl
from jax.experimental.pallas import tpu as pltpu
from jax.experimental.pallas import tpu_sc as plsc

N = 320000
M = 10000
D = 128
MP = 10240

NC, NS = 2, 16
PER_TILE = N // (NC * NS)
CHUNK = 112
NBUF = 3
NFULL = PER_TILE // CHUNK
TAIL = PER_TILE - NFULL * CHUNK
OUT_ROWS = MP // NS
CROWS = 8


def _mm_relu_body(x_ref, w_ref, b_ref, o_ref):
    x = x_ref[...].astype(jnp.bfloat16)
    w = w_ref[...].astype(jnp.bfloat16)
    o_ref[...] = jnp.maximum(
        jnp.dot(x, w, preferred_element_type=jnp.float32) + b_ref[...], 0.0)


def _node_transform(hyper_node, W_v2e, b_v2e):
    BN = 6400
    grid = (N // BN,)
    return pl.pallas_call(
        _mm_relu_body,
        grid=grid,
        in_specs=[
            pl.BlockSpec((BN, D), lambda i: (i, 0)),
            pl.BlockSpec((D, D), lambda i: (0, 0)),
            pl.BlockSpec((1, D), lambda i: (0, 0)),
        ],
        out_specs=pl.BlockSpec((BN, D), lambda i: (i, 0)),
        out_shape=jax.ShapeDtypeStruct((N, D), jnp.float32),
    )(hyper_node, W_v2e, b_v2e.reshape(1, D))


def _scatter_body(ni_hbm, idx_hbm, zsum_hbm,
                  sum_out, cnt_out,
                  rows_v0, rows_v1, rows_v2,
                  idx_v0, idx_v1, idx_v2, idxt_v,
                  ones_v, zc_v, sum_acc, cnt_acc,
                  rsem0, rsem1, rsem2,
                  isem0, isem1, isem2,
                  ssem0, ssem1, ssem2,
                  csem0, csem1, csem2):
    c = lax.axis_index("c")
    s = lax.axis_index("s")
    rows = (rows_v0, rows_v1, rows_v2)
    idxs = (idx_v0, idx_v1, idx_v2)
    rsem = (rsem0, rsem1, rsem2)
    isem = (isem0, isem1, isem2)
    ssem = (ssem0, ssem1, ssem2)
    csem = (csem0, csem1, csem2)

    one16 = jnp.ones((16,), jnp.float32)
    zero16 = jnp.zeros((16,), jnp.float32)
    for k in range(CHUNK // 16):
        ones_v[pl.ds(16 * k, 16)] = one16
    for k in range(OUT_ROWS // 16):
        zc_v[pl.ds(16 * k, 16)] = zero16

    r0 = pl.multiple_of(s * OUT_ROWS, 8)
    pltpu.sync_copy(zsum_hbm.at[pl.ds(r0, OUT_ROWS)],
                    sum_acc.at[pl.ds(r0, OUT_ROWS)])
    pltpu.sync_copy(zc_v, cnt_acc.at[pl.ds(r0, OUT_ROWS)])
    plsc.subcore_barrier()

    node_base = c * (N // NC) + s * PER_TILE

    def start_fetch(k, b):
        kk = jnp.where(k < NFULL, k, 0)
        base = pl.multiple_of(node_base + kk * CHUNK, 8)
        pltpu.make_async_copy(ni_hbm.at[pl.ds(base, CHUNK)],
                              rows[b], rsem[b]).start()
        pltpu.make_async_copy(idx_hbm.at[pl.ds(base, CHUNK)],
                              idxs[b], isem[b]).start()

    def wait_fetch(b):
        pltpu.make_async_copy(ni_hbm.at[pl.ds(0, CHUNK)],
                              rows[b], rsem[b]).wait()
        pltpu.make_async_copy(idx_hbm.at[pl.ds(0, CHUNK)],
                              idxs[b], isem[b]).wait()

    def start_scat(b):
        pltpu.make_async_copy(rows[b], sum_acc.at[idxs[b]],
                              ssem[b]).start(add=True)
        pltpu.make_async_copy(ones_v, cnt_acc.at[idxs[b]],
                              csem[b]).start(add=True)

    def wait_scat(b):
        pltpu.make_async_copy(rows[b], sum_acc.at[idxs[b]], ssem[b]).wait()
        pltpu.make_async_copy(ones_v, cnt_acc.at[idxs[b]], csem[b]).wait()

    def step(k, b, first=False):
        wait_fetch(b)
        start_scat(b)
        if not first:
            wait_scat((b - 1) % NBUF)
        start_fetch(k + 2, (b + 2) % NBUF)

    start_fetch(0, 0)
    start_fetch(1, 1)
    step(0, 0, first=True)

    @pl.loop(0, (NFULL - 2) // NBUF)
    def body(g):
        k = NBUF * g + 1
        step(k, 1)
        step(k + 1, 2)
        step(k + 2, 0)

    rem = (NFULL - 1) - ((NFULL - 2) // NBUF) * NBUF
    for i in range(rem):
        k = NFULL - rem + i
        step(k, k % NBUF)

    wait_scat((NFULL - 1) % NBUF)
    wait_fetch(NFULL % NBUF)
    wait_fetch((NFULL + 1) % NBUF)

    tbase = pl.multiple_of(node_base + NFULL * CHUNK, 8)
    pltpu.sync_copy(ni_hbm.at[pl.ds(tbase, TAIL)], rows_v0.at[pl.ds(0, TAIL)])
    pltpu.sync_copy(idx_hbm.at[pl.ds(tbase, TAIL)], idxt_v)
    pltpu.sync_copy(rows_v0.at[pl.ds(0, TAIL)], sum_acc.at[idxt_v], add=True)
    pltpu.sync_copy(ones_v.at[pl.ds(0, TAIL)], cnt_acc.at[idxt_v], add=True)

    plsc.subcore_barrier()

    pltpu.sync_copy(sum_acc.at[pl.ds(r0, OUT_ROWS)],
                    sum_out.at[c, pl.ds(r0, OUT_ROWS)])
    pltpu.sync_copy(cnt_acc.at[pl.ds(r0, OUT_ROWS)], zc_v)
    for k in range(OUT_ROWS // 16):
        rows_v0[k // 8, pl.ds((k % 8) * 16, 16)] = zc_v[pl.ds(16 * k, 16)]
    cr0 = pl.multiple_of(s * CROWS, 8)
    pltpu.sync_copy(rows_v0.at[pl.ds(0, CROWS)],
                    cnt_out.at[c, pl.ds(cr0, CROWS)])


def _segment_sums(node_info, idx, zsum):
    mesh = plsc.VectorSubcoreMesh(core_axis_name="c", subcore_axis_name="s")
    f = functools.partial(
        pl.kernel,
        out_type=(jax.ShapeDtypeStruct((NC, MP, D), jnp.float32),
                  jax.ShapeDtypeStruct((NC, NS * CROWS, D), jnp.float32)),
        mesh=mesh,
        scratch_types=[
            pltpu.VMEM((CHUNK, D), jnp.float32),
            pltpu.VMEM((CHUNK, D), jnp.float32),
            pltpu.VMEM((CHUNK, D), jnp.float32),
            pltpu.VMEM((CHUNK,), jnp.int32),
            pltpu.VMEM((CHUNK,), jnp.int32),
            pltpu.VMEM((CHUNK,), jnp.int32),
            pltpu.VMEM((TAIL,), jnp.int32),
            pltpu.VMEM((CHUNK,), jnp.float32),
            pltpu.VMEM((OUT_ROWS,), jnp.float32),
            pltpu.VMEM_SHARED((MP, D), jnp.float32),
            pltpu.VMEM_SHARED((MP,), jnp.float32),
        ] + [pltpu.SemaphoreType.DMA for _ in range(12)],
    )(_scatter_body)
    return f(node_info, idx, zsum)


def _update_body(sums_ref, cnt_ref, he_ref, wa_ref, wb_ref, b_ref, o_ref):
    ssum = sums_ref[0] + sums_ref[1]
    cnt = jnp.sum(cnt_ref[...], axis=1, keepdims=True)
    mean = ssum / jnp.maximum(cnt, 1.0)
    h = jnp.dot(mean, wa_ref[...], preferred_element_type=jnp.float32)
    h = h + jnp.dot(he_ref[...], wb_ref[...], preferred_element_type=jnp.float32)
    h = jnp.maximum(h + b_ref[...], 0.0)
    nrm = jnp.sqrt(jnp.sum(h * h, axis=-1, keepdims=True))
    o_ref[...] = h / jnp.maximum(nrm, 1e-12)


def _update(sums, cntsT, hyperedge, W_upd, b_upd):
    BR = 2000
    grid = (M // BR,)
    return pl.pallas_call(
        _update_body,
        grid=grid,
        in_specs=[
            pl.BlockSpec((NC, BR, D), lambda i: (0, i, 0)),
            pl.BlockSpec((BR, NC), lambda i: (i, 0)),
            pl.BlockSpec((BR, D), lambda i: (i, 0)),
            pl.BlockSpec((D, D), lambda i: (0, 0)),
            pl.BlockSpec((D, D), lambda i: (0, 0)),
            pl.BlockSpec((1, D), lambda i: (0, 0)),
        ],
        out_specs=pl.BlockSpec((BR, D), lambda i: (i, 0)),
        out_shape=jax.ShapeDtypeStruct((M, D), jnp.float32),
    )(sums, cntsT, hyperedge, W_upd[:D], W_upd[D:], b_upd.reshape(1, D))


def kernel(hyperedge, hyper_node, ve_affiliation, W_v2e, b_v2e, W_upd, b_upd):
    assert hyper_node.shape == (N, D) and hyperedge.shape == (M, D)
    node_info = _node_transform(hyper_node, W_v2e, b_v2e)
    idx = ve_affiliation[0]
    zsum = jnp.zeros((MP, D), jnp.float32)
    sums, cnts_wide = _segment_sums(node_info, idx, zsum)
    cntsT = (cnts_wide.reshape(NC, NS, CROWS * D)[:, :, :OUT_ROWS]
             .reshape(NC, MP).T)
    return _update(sums, cntsT, hyperedge, W_upd, b_upd)


# speedup vs baseline: 1.0717x; 1.0031x over previous
import functools

import jax
import jax.numpy as jnp
from jax import lax
from jax.experimental import pallas as pl
from jax.experimental.pallas import tpu as pltpu
from jax.experimental.pallas import tpu_sc as plsc

N = 320000
M = 10000
D = 128
MP = 10240

NC, NS = 2, 16
NSPLIT = 2
NH = N // NSPLIT
PER_TILE = NH // (NC * NS)
CHUNK = 112
NBUF = 3
NFULL = PER_TILE // CHUNK
TAIL = PER_TILE - NFULL * CHUNK
OUT_ROWS = MP // NS
CROWS = 8


def _mm_relu_body(x_ref, w_ref, b_ref, o_ref):
    x = x_ref[...].astype(jnp.bfloat16)
    w = w_ref[...].astype(jnp.bfloat16)
    o_ref[...] = jnp.maximum(
        jnp.dot(x, w, preferred_element_type=jnp.float32) + b_ref[...], 0.0)


def _node_transform(hyper_node, W_v2e, b_v2e, half):
    BN = 6400
    grid = (NH // BN,)
    off = half * (NH // BN)
    return pl.pallas_call(
        _mm_relu_body,
        grid=grid,
        in_specs=[
            pl.BlockSpec((BN, D), lambda i: (i + off, 0)),
            pl.BlockSpec((D, D), lambda i: (0, 0)),
            pl.BlockSpec((1, D), lambda i: (0, 0)),
        ],
        out_specs=pl.BlockSpec((BN, D), lambda i: (i, 0)),
        out_shape=jax.ShapeDtypeStruct((NH, D), jnp.float32),
    )(hyper_node, W_v2e, b_v2e.reshape(1, D))


def _scatter_body(ni_hbm, idx_hbm, zsum_hbm,
                  sum_out, cnt_out,
                  rows_v0, rows_v1, rows_v2,
                  idx_v0, idx_v1, idx_v2, idxt_v,
                  ones_v, zc_v, sum_acc, cnt_acc,
                  rsem0, rsem1, rsem2,
                  isem0, isem1, isem2,
                  ssem0, ssem1, ssem2,
                  csem0, csem1, csem2):
    c = lax.axis_index("c")
    s = lax.axis_index("s")
    rows = (rows_v0, rows_v1, rows_v2)
    idxs = (idx_v0, idx_v1, idx_v2)
    rsem = (rsem0, rsem1, rsem2)
    isem = (isem0, isem1, isem2)
    ssem = (ssem0, ssem1, ssem2)
    csem = (csem0, csem1, csem2)

    one16 = jnp.ones((16,), jnp.float32)
    zero16 = jnp.zeros((16,), jnp.float32)
    for k in range(CHUNK // 16):
        ones_v[pl.ds(16 * k, 16)] = one16
    for k in range(OUT_ROWS // 16):
        zc_v[pl.ds(16 * k, 16)] = zero16

    r0 = pl.multiple_of(s * OUT_ROWS, 8)
    pltpu.sync_copy(zsum_hbm.at[pl.ds(r0, OUT_ROWS)],
                    sum_acc.at[pl.ds(r0, OUT_ROWS)])
    pltpu.sync_copy(zc_v, cnt_acc.at[pl.ds(r0, OUT_ROWS)])
    plsc.subcore_barrier()

    node_base = c * (NH // NC) + s * PER_TILE

    def start_fetch(k, b):
        kk = jnp.where(k < NFULL, k, 0)
        base = pl.multiple_of(node_base + kk * CHUNK, 8)
        pltpu.make_async_copy(ni_hbm.at[pl.ds(base, CHUNK)],
                              rows[b], rsem[b]).start()
        pltpu.make_async_copy(idx_hbm.at[pl.ds(base, CHUNK)],
                              idxs[b], isem[b]).start()

    def wait_fetch(b):
        pltpu.make_async_copy(ni_hbm.at[pl.ds(0, CHUNK)],
                              rows[b], rsem[b]).wait()
        pltpu.make_async_copy(idx_hbm.at[pl.ds(0, CHUNK)],
                              idxs[b], isem[b]).wait()

    def start_scat(b):
        pltpu.make_async_copy(rows[b], sum_acc.at[idxs[b]],
                              ssem[b]).start(add=True)
        pltpu.make_async_copy(ones_v, cnt_acc.at[idxs[b]],
                              csem[b]).start(add=True)

    def wait_scat(b):
        pltpu.make_async_copy(rows[b], sum_acc.at[idxs[b]], ssem[b]).wait()
        pltpu.make_async_copy(ones_v, cnt_acc.at[idxs[b]], csem[b]).wait()

    def step(k, b, first=False):
        wait_fetch(b)
        start_scat(b)
        if not first:
            wait_scat((b - 1) % NBUF)
        start_fetch(k + 2, (b + 2) % NBUF)

    start_fetch(0, 0)
    start_fetch(1, 1)
    step(0, 0, first=True)

    @pl.loop(0, (NFULL - 2) // NBUF)
    def body(g):
        k = NBUF * g + 1
        step(k, 1)
        step(k + 1, 2)
        step(k + 2, 0)

    rem = (NFULL - 1) - ((NFULL - 2) // NBUF) * NBUF
    for i in range(rem):
        k = NFULL - rem + i
        step(k, k % NBUF)

    wait_scat((NFULL - 1) % NBUF)
    wait_fetch(NFULL % NBUF)
    wait_fetch((NFULL + 1) % NBUF)

    tbase = pl.multiple_of(node_base + NFULL * CHUNK, 8)
    pltpu.sync_copy(ni_hbm.at[pl.ds(tbase, TAIL)], rows_v0.at[pl.ds(0, TAIL)])
    pltpu.sync_copy(idx_hbm.at[pl.ds(tbase, TAIL)], idxt_v)
    pltpu.sync_copy(rows_v0.at[pl.ds(0, TAIL)], sum_acc.at[idxt_v], add=True)
    pltpu.sync_copy(ones_v.at[pl.ds(0, TAIL)], cnt_acc.at[idxt_v], add=True)

    plsc.subcore_barrier()

    pltpu.sync_copy(sum_acc.at[pl.ds(r0, OUT_ROWS)],
                    sum_out.at[c, pl.ds(r0, OUT_ROWS)])
    pltpu.sync_copy(cnt_acc.at[pl.ds(r0, OUT_ROWS)], zc_v)
    for k in range(OUT_ROWS // 16):
        rows_v0[k // 8, pl.ds((k % 8) * 16, 16)] = zc_v[pl.ds(16 * k, 16)]
    cr0 = pl.multiple_of(s * CROWS, 8)
    pltpu.sync_copy(rows_v0.at[pl.ds(0, CROWS)],
                    cnt_out.at[c, pl.ds(cr0, CROWS)])


def _segment_sums(node_info, idx, zsum):
    mesh = plsc.VectorSubcoreMesh(core_axis_name="c", subcore_axis_name="s")
    f = functools.partial(
        pl.kernel,
        out_type=(jax.ShapeDtypeStruct((NC, MP, D), jnp.float32),
                  jax.ShapeDtypeStruct((NC, NS * CROWS, D), jnp.float32)),
        mesh=mesh,
        scratch_types=[
            pltpu.VMEM((CHUNK, D), jnp.float32),
            pltpu.VMEM((CHUNK, D), jnp.float32),
            pltpu.VMEM((CHUNK, D), jnp.float32),
            pltpu.VMEM((CHUNK,), jnp.int32),
            pltpu.VMEM((CHUNK,), jnp.int32),
            pltpu.VMEM((CHUNK,), jnp.int32),
            pltpu.VMEM((TAIL,), jnp.int32),
            pltpu.VMEM((CHUNK,), jnp.float32),
            pltpu.VMEM((OUT_ROWS,), jnp.float32),
            pltpu.VMEM_SHARED((MP, D), jnp.float32),
            pltpu.VMEM_SHARED((MP,), jnp.float32),
        ] + [pltpu.SemaphoreType.DMA for _ in range(12)],
    )(_scatter_body)
    return f(node_info, idx, zsum)


def _update_body(sa_ref, sb_ref, cnt_ref, he_ref, wa_ref, wb_ref, b_ref,
                 o_ref):
    ssum = sa_ref[0] + sa_ref[1] + sb_ref[0] + sb_ref[1]
    cnt = jnp.sum(cnt_ref[...], axis=1, keepdims=True)
    mean = ssum / jnp.maximum(cnt, 1.0)
    h = jnp.dot(mean, wa_ref[...], preferred_element_type=jnp.float32)
    h = h + jnp.dot(he_ref[...], wb_ref[...], preferred_element_type=jnp.float32)
    h = jnp.maximum(h + b_ref[...], 0.0)
    nrm = jnp.sqrt(jnp.sum(h * h, axis=-1, keepdims=True))
    o_ref[...] = h / jnp.maximum(nrm, 1e-12)


def _update(sums_a, sums_b, cntsT, hyperedge, W_upd, b_upd):
    BR = 2000
    grid = (M // BR,)
    return pl.pallas_call(
        _update_body,
        grid=grid,
        in_specs=[
            pl.BlockSpec((NC, BR, D), lambda i: (0, i, 0)),
            pl.BlockSpec((NC, BR, D), lambda i: (0, i, 0)),
            pl.BlockSpec((BR, NSPLIT * NC), lambda i: (i, 0)),
            pl.BlockSpec((BR, D), lambda i: (i, 0)),
            pl.BlockSpec((D, D), lambda i: (0, 0)),
            pl.BlockSpec((D, D), lambda i: (0, 0)),
            pl.BlockSpec((1, D), lambda i: (0, 0)),
        ],
        out_specs=pl.BlockSpec((BR, D), lambda i: (i, 0)),
        out_shape=jax.ShapeDtypeStruct((M, D), jnp.float32),
    )(sums_a, sums_b, cntsT, hyperedge, W_upd[:D], W_upd[D:],
      b_upd.reshape(1, D))


def kernel(hyperedge, hyper_node, ve_affiliation, W_v2e, b_v2e, W_upd, b_upd):
    assert hyper_node.shape == (N, D) and hyperedge.shape == (M, D)
    idx = ve_affiliation[0]
    zsum = jnp.zeros((MP, D), jnp.float32)
    ni_a = _node_transform(hyper_node, W_v2e, b_v2e, 0)
    ni_b = _node_transform(hyper_node, W_v2e, b_v2e, 1)
    sums_a, cw_a = _segment_sums(ni_a, idx[:NH], zsum)
    sums_b, cw_b = _segment_sums(ni_b, idx[NH:], zsum)

    def counts(cw):
        return cw.reshape(NC, NS, CROWS * D)[:, :, :OUT_ROWS].reshape(NC, MP)

    cntsT = jnp.concatenate([counts(cw_a), counts(cw_b)], axis=0).T
    return _update(sums_a, sums_b, cntsT, hyperedge, W_upd, b_upd)
